# baseline (speedup 1.0000x reference)
import functools

import jax
import jax.numpy as jnp
from jax import lax
from jax.experimental import pallas as pl
from jax.experimental.pallas import tpu as pltpu
from jax.experimental.pallas import tpu_sc as plsc

_B = 64
_V = 100000
_NS = 5
_K = 5
_EPS = 1.0

_NC = 2
_NSUB = 16
_NW = _NC * _NSUB
_L = 16
_KK = _K + 1

_NBLK = 8
_NQ = 4
_RPB = _B // _NBLK
_VC = 1024
_CPQ = 24
_VMAIN = _NQ * _CPQ * _VC
_TAILV = _VMAIN + _VC
_TAILN = _V - _TAILV
_NIT = _VC // _L
_NBK = _NIT // 2
_NST = _NS * _RPB * _KK * _L

_NEG = float("-inf")


def _iota16():
    return lax.iota(jnp.int32, _L)


def _vec_i32(x):
    return _iota16() * 0 + x


def _sc_partials(s2d, y, zt3, s_tail, z_tail):
    mesh = plsc.VectorSubcoreMesh(core_axis_name="c", subcore_axis_name="s")

    @functools.partial(
        pl.kernel,
        out_type=(
            jax.ShapeDtypeStruct((_NW * _NST,), jnp.float32),
            jax.ShapeDtypeStruct((_NW * _L,), jnp.float32),
        ),
        mesh=mesh,
        compiler_params=pltpu.CompilerParams(needs_layout_passes=False),
        scratch_types=[
            pltpu.VMEM((_NS, _RPB, _VC), jnp.float32),
            pltpu.VMEM((_NS, _RPB, _VC), jnp.float32),
            pltpu.VMEM((_RPB, _VC), jnp.float32),
            pltpu.VMEM((_RPB, _VC), jnp.float32),
            pltpu.VMEM((_NST,), jnp.float32),
            pltpu.VMEM((_B,), jnp.int32),
            pltpu.VMEM((_L,), jnp.float32),
            pltpu.SemaphoreType.DMA,
            pltpu.SemaphoreType.DMA,
        ],
    )
    def body(s_hbm, y_hbm, z_hbm, st_hbm, zt_hbm, cand_hbm, corr_hbm,
             zbuf0, zbuf1, sbuf0, sbuf1, st, ybuf, cbuf, sem0, sem1):
        wid = lax.axis_index("s") * _NC + lax.axis_index("c")
        iota = _iota16()
        bufs = ((zbuf0, sbuf0, sem0), (zbuf1, sbuf1, sem1))

        blk = wid // _NQ
        q = wid % _NQ
        row0 = pl.multiple_of(blk * _RPB, 8)
        c_lo = q * _CPQ

        pltpu.sync_copy(y_hbm, ybuf)
        yv8 = plsc.load_gather(ybuf, [row0 + (iota & (_RPB - 1))])
        lane_ok = iota < _RPB

        vneg = jnp.full((_L,), _NEG, jnp.float32)

        def init_st(t, _):
            st[pl.ds(t * _L, _L)] = vneg
            return 0

        lax.fori_loop(0, _NST // _L, init_st, 0)

        def copies(c, zb, sb, zsrc, ssrc):
            v0 = pl.multiple_of(c * _VC, 128)
            out = []
            for j in range(_NS):
                out.append((zsrc.at[j, pl.ds(row0, _RPB), pl.ds(v0, _VC)],
                            zb.at[j]))
            out.append((ssrc.at[pl.ds(row0, _RPB), pl.ds(v0, _VC)], sb))
            return out

        def start(c, zb, sb, sem, zsrc=None, ssrc=None):
            zsrc = z_hbm if zsrc is None else zsrc
            ssrc = s_hbm if ssrc is None else ssrc
            for src, dst in copies(c, zb, sb, zsrc, ssrc):
                pltpu.async_copy(src, dst, sem)

        def wait(c, zb, sb, sem, zsrc=None, ssrc=None):
            zsrc = z_hbm if zsrc is None else zsrc
            ssrc = s_hbm if ssrc is None else ssrc
            for src, dst in copies(c, zb, sb, zsrc, ssrc):
                pltpu.make_async_copy(src, dst, sem).wait()

        def corr_update(sb, base, corr):
            off = yv8 - base
            inr = jnp.logical_and(off >= 0, off < _VC)
            offc = jnp.clip(off, 0, _VC - 1)
            g = plsc.load_gather(sb, [iota & (_RPB - 1), offc])
            take = jnp.logical_and(inr, lane_ok)
            return corr + jnp.where(take, g, 0.0)

        def compute_chunk(zb, sb):
            def pval(i, v, j, zb=zb, sb=sb):
                sv = sb[i, pl.ds(v * _L, _L)]
                zv = zb[j, i, pl.ds(v * _L, _L)]
                return sv + zv if _EPS == 1.0 else sv + _EPS * zv

            def rbody(i, _, zb=zb, sb=sb):
                sbase = i * (_KK * _NS * _L)
                ms = []
                for j in range(_NS):
                    for r in range(_KK):
                        ms.append(st[pl.ds(sbase + (j * _KK + r) * _L, _L)])

                def inner(vb, ms, zb=zb, sb=sb, i=i):
                    d = None
                    for u in range(2):
                        for j in range(_NS):
                            p = pval(i, vb * 2 + u, j)
                            x = p - ms[j * _KK + _KK - 1]
                            d = x if d is None else jnp.maximum(d, x)
                    pred = jnp.max(d) > 0.0

                    def ins(ms, i=i, vb=vb):
                        out = list(ms)
                        for u in range(2):
                            for j in range(_NS):
                                p = pval(i, vb * 2 + u, j)
                                for r in range(_KK):
                                    hi = jnp.maximum(out[j * _KK + r], p)
                                    p = jnp.minimum(out[j * _KK + r], p)
                                    out[j * _KK + r] = hi
                        return tuple(out)

                    return lax.cond(pred, ins, lambda ms: ms, tuple(ms))

                ms = lax.fori_loop(0, _NBK, inner, tuple(ms))
                for j in range(_NS):
                    for r in range(_KK):
                        st[pl.ds(sbase + (j * _KK + r) * _L, _L)] = \
                            ms[j * _KK + r]
                return 0

            lax.fori_loop(0, _RPB, rbody, 0)

        for half in range(2):
            start(c_lo + half, bufs[half][0], bufs[half][1], bufs[half][2])

        def g_body(g, corr):
            for half in range(2):
                zb, sb, sem = bufs[half]
                c = c_lo + g * 2 + half
                wait(c, zb, sb, sem)
                corr = corr_update(sb, c * _VC, corr)
                compute_chunk(zb, sb)

                @pl.when(g < _CPQ // 2 - 1)
                def _(c=c, zb=zb, sb=sb, sem=sem):
                    start(c + 2, zb, sb, sem)

            return corr

        corr = lax.fori_loop(0, _CPQ // 2, g_body,
                             jnp.full((_L,), 0.0, jnp.float32))

        @pl.when(q == 0)
        def _():
            zb, sb, sem = bufs[0]
            ce = _VMAIN // _VC
            start(ce, zb, sb, sem)
            wait(ce, zb, sb, sem)
            c2 = corr_update(sb, ce * _VC, corr)
            compute_chunk(zb, sb)
            cbuf[...] = c2
            pltpu.sync_copy(cbuf, corr_hbm.at[pl.ds(wid * _L, _L)])

        @pl.when(q == 1)
        def _():
            zb, sb, sem = bufs[0]
            start(0, zb, sb, sem, zsrc=zt_hbm, ssrc=st_hbm)
            wait(0, zb, sb, sem, zsrc=zt_hbm, ssrc=st_hbm)
            c2 = corr_update(sb, _TAILV, corr)
            compute_chunk(zb, sb)
            cbuf[...] = c2
            pltpu.sync_copy(cbuf, corr_hbm.at[pl.ds(wid * _L, _L)])

        @pl.when(q >= 2)
        def _():
            cbuf[...] = corr
            pltpu.sync_copy(cbuf, corr_hbm.at[pl.ds(wid * _L, _L)])

        pltpu.sync_copy(st, cand_hbm.at[pl.ds(wid * _NST, _NST)])

    return body(s2d, y, zt3, s_tail, z_tail)


def _tc_loss(cand2d, corr2d):
    def fin(x_ref, c_ref, o_ref):
        x = x_ref[...]
        need = jnp.full((_NS * _B, 1), _KK, jnp.int32)
        ans = jnp.zeros((_NS * _B, 1), jnp.float32)
        t = jnp.full((_NS * _B, 1), jnp.inf, jnp.float32)
        for _ in range(_KK):
            masked = jnp.where(x < t, x, _NEG)
            m = jnp.max(masked, axis=1, keepdims=True)
            c = jnp.sum((x == m).astype(jnp.int32), axis=1, keepdims=True)
            hit = jnp.logical_and(need > 0, need <= c)
            ans = jnp.where(hit, m, ans)
            need = need - c
            t = m
        skp1 = jnp.zeros((_B, 1), jnp.float32)
        for j in range(_NS):
            skp1 = skp1 + ans[j * _B:(j + 1) * _B, :]
        skp1 = skp1 * jnp.float32(1.0 / _NS)
        corr = jnp.sum(c_ref[...], axis=1, keepdims=True)
        num = jnp.maximum(jnp.float32(1.0) + skp1 - corr, 0.0)
        o_ref[0, 0] = jnp.sum(num) * jnp.float32(1.0 / _B)

    return pl.pallas_call(
        fin,
        out_shape=jax.ShapeDtypeStruct((1, 1), jnp.float32),
        out_specs=pl.BlockSpec(memory_space=pltpu.SMEM),
    )(cand2d, corr2d)


def kernel(s, y, Z):
    zt3 = jnp.transpose(Z, (2, 0, 1))
    pad = _VC - _TAILN
    s_tail = jnp.pad(s[:, _TAILV:], ((0, 0), (0, pad)))
    z_tail = jnp.pad(zt3[:, :, _TAILV:], ((0, 0), (0, 0), (0, pad)),
                     constant_values=_NEG)
    cand, corr = _sc_partials(s, y.astype(jnp.int32), zt3, s_tail, z_tail)
    cand2d = cand.reshape(_NBLK, _NQ, _RPB, _NS, _KK * _L)
    cand2d = cand2d.transpose(3, 0, 2, 1, 4).reshape(_NS * _B, _NQ * _KK * _L)
    corr2d = (corr.reshape(_NBLK, _NQ, _L)[:, :, :_RPB]
              .transpose(0, 2, 1).reshape(_B, _NQ))
    return _tc_loss(cand2d, corr2d)[0, 0]


# speedup vs baseline: 1.8403x; 1.8403x over previous
import functools

import jax
import jax.numpy as jnp
from jax import lax
from jax.experimental import pallas as pl
from jax.experimental.pallas import tpu as pltpu
from jax.experimental.pallas import tpu_sc as plsc

_B = 64
_V = 100000
_NS = 5
_K = 5
_EPS = 1.0

_NC = 2
_NSUB = 16
_NW = _NC * _NSUB
_L = 16
_KK = _K + 1

_NBLK = 8
_NQ = 4
_RPB = _B // _NBLK
_VC = 1024
_CPQ = 24
_VMAIN = _NQ * _CPQ * _VC
_TAILV = _VMAIN + _VC
_TAILN = _V - _TAILV
_NIT = _VC // _L
_NBK = _NIT // 2
_NST = _NS * _RPB * _KK * _L

_NEG = float("-inf")


def _iota16():
    return lax.iota(jnp.int32, _L)


def _vec_i32(x):
    return _iota16() * 0 + x


def _sc_partials(s2d, y, zt3, s_tail, z_tail):
    mesh = plsc.VectorSubcoreMesh(core_axis_name="c", subcore_axis_name="s")

    @functools.partial(
        pl.kernel,
        out_type=(
            jax.ShapeDtypeStruct((_NW * _NST,), jnp.float32),
            jax.ShapeDtypeStruct((_NW * _L,), jnp.float32),
        ),
        mesh=mesh,
        compiler_params=pltpu.CompilerParams(needs_layout_passes=False),
        scratch_types=[
            pltpu.VMEM((_NS, _RPB, _VC), jnp.float32),
            pltpu.VMEM((_NS, _RPB, _VC), jnp.float32),
            pltpu.VMEM((_RPB, _VC), jnp.float32),
            pltpu.VMEM((_RPB, _VC), jnp.float32),
            pltpu.VMEM((_NST,), jnp.float32),
            pltpu.VMEM((_B,), jnp.int32),
            pltpu.VMEM((_L,), jnp.float32),
            pltpu.SemaphoreType.DMA,
            pltpu.SemaphoreType.DMA,
        ],
    )
    def body(s_hbm, y_hbm, z_hbm, st_hbm, zt_hbm, cand_hbm, corr_hbm,
             zbuf0, zbuf1, sbuf0, sbuf1, st, ybuf, cbuf, sem0, sem1):
        wid = lax.axis_index("s") * _NC + lax.axis_index("c")
        iota = _iota16()
        bufs = ((zbuf0, sbuf0, sem0), (zbuf1, sbuf1, sem1))

        blk = wid // _NQ
        q = wid % _NQ
        row0 = pl.multiple_of(blk * _RPB, 8)
        c_lo = q * _CPQ

        pltpu.sync_copy(y_hbm, ybuf)
        yv8 = plsc.load_gather(ybuf, [row0 + (iota & (_RPB - 1))])
        lane_ok = iota < _RPB

        vneg = jnp.full((_L,), _NEG, jnp.float32)

        def init_st(t, _):
            st[pl.ds(t * _L, _L)] = vneg
            return 0

        lax.fori_loop(0, _NST // _L, init_st, 0)

        def copies(c, zb, sb, zsrc, ssrc):
            v0 = pl.multiple_of(c * _VC, 128)
            out = []
            for j in range(_NS):
                out.append((zsrc.at[j, pl.ds(row0, _RPB), pl.ds(v0, _VC)],
                            zb.at[j]))
            out.append((ssrc.at[pl.ds(row0, _RPB), pl.ds(v0, _VC)], sb))
            return out

        def start(c, zb, sb, sem, zsrc=None, ssrc=None):
            zsrc = z_hbm if zsrc is None else zsrc
            ssrc = s_hbm if ssrc is None else ssrc
            for src, dst in copies(c, zb, sb, zsrc, ssrc):
                pltpu.async_copy(src, dst, sem)

        def wait(c, zb, sb, sem, zsrc=None, ssrc=None):
            zsrc = z_hbm if zsrc is None else zsrc
            ssrc = s_hbm if ssrc is None else ssrc
            for src, dst in copies(c, zb, sb, zsrc, ssrc):
                pltpu.make_async_copy(src, dst, sem).wait()

        def corr_update(sb, base, corr):
            off = yv8 - base
            inr = jnp.logical_and(off >= 0, off < _VC)
            offc = jnp.clip(off, 0, _VC - 1)
            g = plsc.load_gather(sb, [iota & (_RPB - 1), offc])
            take = jnp.logical_and(inr, lane_ok)
            return corr + jnp.where(take, g, 0.0)

        def compute_chunk(zb, sb):
            def pval(i, v, j, zb=zb, sb=sb):
                sv = sb[i, pl.ds(v * _L, _L)]
                zv = zb[j, i, pl.ds(v * _L, _L)]
                return sv + zv if _EPS == 1.0 else sv + _EPS * zv

            def rbody(i, _, zb=zb, sb=sb):
                sbase = i * (_KK * _NS * _L)
                ms = []
                for j in range(_NS):
                    for r in range(_KK):
                        ms.append(st[pl.ds(sbase + (j * _KK + r) * _L, _L)])

                def inner(v, ms, zb=zb, sb=sb, i=i):
                    out = list(ms)
                    for j in range(_NS):
                        p = pval(i, v, j)
                        for r in range(_KK):
                            hi = jnp.maximum(out[j * _KK + r], p)
                            p = jnp.minimum(out[j * _KK + r], p)
                            out[j * _KK + r] = hi
                    return tuple(out)

                ms = lax.fori_loop(0, _NIT, inner, tuple(ms))
                for j in range(_NS):
                    for r in range(_KK):
                        st[pl.ds(sbase + (j * _KK + r) * _L, _L)] = \
                            ms[j * _KK + r]
                return 0

            lax.fori_loop(0, _RPB, rbody, 0)

        for half in range(2):
            start(c_lo + half, bufs[half][0], bufs[half][1], bufs[half][2])

        def g_body(g, corr):
            for half in range(2):
                zb, sb, sem = bufs[half]
                c = c_lo + g * 2 + half
                wait(c, zb, sb, sem)
                corr = corr_update(sb, c * _VC, corr)
                compute_chunk(zb, sb)

                @pl.when(g < _CPQ // 2 - 1)
                def _(c=c, zb=zb, sb=sb, sem=sem):
                    start(c + 2, zb, sb, sem)

            return corr

        corr = lax.fori_loop(0, _CPQ // 2, g_body,
                             jnp.full((_L,), 0.0, jnp.float32))

        @pl.when(q == 0)
        def _():
            zb, sb, sem = bufs[0]
            ce = _VMAIN // _VC
            start(ce, zb, sb, sem)
            wait(ce, zb, sb, sem)
            c2 = corr_update(sb, ce * _VC, corr)
            compute_chunk(zb, sb)
            cbuf[...] = c2
            pltpu.sync_copy(cbuf, corr_hbm.at[pl.ds(wid * _L, _L)])

        @pl.when(q == 1)
        def _():
            zb, sb, sem = bufs[0]
            start(0, zb, sb, sem, zsrc=zt_hbm, ssrc=st_hbm)
            wait(0, zb, sb, sem, zsrc=zt_hbm, ssrc=st_hbm)
            c2 = corr_update(sb, _TAILV, corr)
            compute_chunk(zb, sb)
            cbuf[...] = c2
            pltpu.sync_copy(cbuf, corr_hbm.at[pl.ds(wid * _L, _L)])

        @pl.when(q >= 2)
        def _():
            cbuf[...] = corr
            pltpu.sync_copy(cbuf, corr_hbm.at[pl.ds(wid * _L, _L)])

        pltpu.sync_copy(st, cand_hbm.at[pl.ds(wid * _NST, _NST)])

    return body(s2d, y, zt3, s_tail, z_tail)


def _tc_loss(cand2d, corr2d):
    def fin(x_ref, c_ref, o_ref):
        x = x_ref[...]
        need = jnp.full((_NS * _B, 1), _KK, jnp.int32)
        ans = jnp.zeros((_NS * _B, 1), jnp.float32)
        t = jnp.full((_NS * _B, 1), jnp.inf, jnp.float32)
        for _ in range(_KK):
            masked = jnp.where(x < t, x, _NEG)
            m = jnp.max(masked, axis=1, keepdims=True)
            c = jnp.sum((x == m).astype(jnp.int32), axis=1, keepdims=True)
            hit = jnp.logical_and(need > 0, need <= c)
            ans = jnp.where(hit, m, ans)
            need = need - c
            t = m
        skp1 = jnp.zeros((_B, 1), jnp.float32)
        for j in range(_NS):
            skp1 = skp1 + ans[j * _B:(j + 1) * _B, :]
        skp1 = skp1 * jnp.float32(1.0 / _NS)
        corr = jnp.sum(c_ref[...], axis=1, keepdims=True)
        num = jnp.maximum(jnp.float32(1.0) + skp1 - corr, 0.0)
        o_ref[0, 0] = jnp.sum(num) * jnp.float32(1.0 / _B)

    return pl.pallas_call(
        fin,
        out_shape=jax.ShapeDtypeStruct((1, 1), jnp.float32),
        out_specs=pl.BlockSpec(memory_space=pltpu.SMEM),
    )(cand2d, corr2d)


def kernel(s, y, Z):
    zt3 = jnp.transpose(Z, (2, 0, 1))
    pad = _VC - _TAILN
    s_tail = jnp.pad(s[:, _TAILV:], ((0, 0), (0, pad)))
    z_tail = jnp.pad(zt3[:, :, _TAILV:], ((0, 0), (0, 0), (0, pad)),
                     constant_values=_NEG)
    cand, corr = _sc_partials(s, y.astype(jnp.int32), zt3, s_tail, z_tail)
    cand2d = cand.reshape(_NBLK, _NQ, _RPB, _NS, _KK * _L)
    cand2d = cand2d.transpose(3, 0, 2, 1, 4).reshape(_NS * _B, _NQ * _KK * _L)
    corr2d = (corr.reshape(_NBLK, _NQ, _L)[:, :, :_RPB]
              .transpose(0, 2, 1).reshape(_B, _NQ))
    return _tc_loss(cand2d, corr2d)[0, 0]


# speedup vs baseline: 1.9130x; 1.0395x over previous
import functools

import jax
import jax.numpy as jnp
from jax import lax
from jax.experimental import pallas as pl
from jax.experimental.pallas import tpu as pltpu
from jax.experimental.pallas import tpu_sc as plsc

_B = 64
_V = 100000
_NS = 5
_K = 5
_EPS = 1.0

_NC = 2
_NSUB = 16
_NW = _NC * _NSUB
_L = 16
_KK = _K + 1

_NBLK = 8
_NQ = 4
_RPB = _B // _NBLK
_VC = 1024
_CPQ = 24
_VMAIN = _NQ * _CPQ * _VC
_TAILV = _VMAIN + _VC
_TAILN = _V - _TAILV
_NIT = _VC // _L
_NBK = _NIT // 2
_NST = _NS * _RPB * _KK * _L

_NEG = float("-inf")


def _iota16():
    return lax.iota(jnp.int32, _L)


def _vec_i32(x):
    return _iota16() * 0 + x


def _sc_partials(s2d, y, zt3, s_tail, z_tail):
    mesh = plsc.VectorSubcoreMesh(core_axis_name="c", subcore_axis_name="s")

    @functools.partial(
        pl.kernel,
        out_type=(
            jax.ShapeDtypeStruct((_NW * _NST,), jnp.float32),
            jax.ShapeDtypeStruct((_NW * _L,), jnp.float32),
        ),
        mesh=mesh,
        compiler_params=pltpu.CompilerParams(needs_layout_passes=False),
        scratch_types=[
            pltpu.VMEM((_NS, _RPB, _VC), jnp.float32),
            pltpu.VMEM((_NS, _RPB, _VC), jnp.float32),
            pltpu.VMEM((_RPB, _VC), jnp.float32),
            pltpu.VMEM((_RPB, _VC), jnp.float32),
            pltpu.VMEM((_NST,), jnp.float32),
            pltpu.VMEM((_B,), jnp.int32),
            pltpu.VMEM((_L,), jnp.float32),
            pltpu.SemaphoreType.DMA,
            pltpu.SemaphoreType.DMA,
        ],
    )
    def body(s_hbm, y_hbm, z_hbm, st_hbm, zt_hbm, cand_hbm, corr_hbm,
             zbuf0, zbuf1, sbuf0, sbuf1, st, ybuf, cbuf, sem0, sem1):
        wid = lax.axis_index("s") * _NC + lax.axis_index("c")
        iota = _iota16()
        bufs = ((zbuf0, sbuf0, sem0), (zbuf1, sbuf1, sem1))

        blk = wid // _NQ
        q = wid % _NQ
        row0 = pl.multiple_of(blk * _RPB, 8)
        c_lo = q * _CPQ

        pltpu.sync_copy(y_hbm, ybuf)
        yv8 = plsc.load_gather(ybuf, [row0 + (iota & (_RPB - 1))])
        lane_ok = iota < _RPB

        vneg = jnp.full((_L,), _NEG, jnp.float32)

        def init_st(t, _):
            st[pl.ds(t * _L, _L)] = vneg
            return 0

        lax.fori_loop(0, _NST // _L, init_st, 0)

        def copies(c, zb, sb, zsrc, ssrc):
            v0 = pl.multiple_of(c * _VC, 128)
            out = []
            for j in range(_NS):
                out.append((zsrc.at[j, pl.ds(row0, _RPB), pl.ds(v0, _VC)],
                            zb.at[j]))
            out.append((ssrc.at[pl.ds(row0, _RPB), pl.ds(v0, _VC)], sb))
            return out

        def start(c, zb, sb, sem, zsrc=None, ssrc=None):
            zsrc = z_hbm if zsrc is None else zsrc
            ssrc = s_hbm if ssrc is None else ssrc
            for src, dst in copies(c, zb, sb, zsrc, ssrc):
                pltpu.async_copy(src, dst, sem)

        def wait(c, zb, sb, sem, zsrc=None, ssrc=None):
            zsrc = z_hbm if zsrc is None else zsrc
            ssrc = s_hbm if ssrc is None else ssrc
            for src, dst in copies(c, zb, sb, zsrc, ssrc):
                pltpu.make_async_copy(src, dst, sem).wait()

        def corr_update(sb, base, corr):
            off = yv8 - base
            inr = jnp.logical_and(off >= 0, off < _VC)
            offc = jnp.clip(off, 0, _VC - 1)
            g = plsc.load_gather(sb, [iota & (_RPB - 1), offc])
            take = jnp.logical_and(inr, lane_ok)
            return corr + jnp.where(take, g, 0.0)

        def compute_chunk(zb, sb):
            def pval(i, v, j, zb=zb, sb=sb):
                sv = sb[i, pl.ds(v * _L, _L)]
                zv = zb[j, i, pl.ds(v * _L, _L)]
                return sv + zv if _EPS == 1.0 else sv + _EPS * zv

            def rbody(i, _, zb=zb, sb=sb):
                sbase = i * (_KK * _NS * _L)
                ms = []
                for j in range(_NS):
                    for r in range(_KK):
                        ms.append(st[pl.ds(sbase + (j * _KK + r) * _L, _L)])

                def inner(v, ms, zb=zb, sb=sb, i=i):
                    out = list(ms)
                    for j in range(_NS):
                        p = pval(i, v, j)
                        for r in range(_KK):
                            hi = jnp.maximum(out[j * _KK + r], p)
                            p = jnp.minimum(out[j * _KK + r], p)
                            out[j * _KK + r] = hi
                    return tuple(out)

                ms = plsc.parallel_loop(0, _NIT, 1, unroll=2,
                                        carry=tuple(ms))(inner)
                for j in range(_NS):
                    for r in range(_KK):
                        st[pl.ds(sbase + (j * _KK + r) * _L, _L)] = \
                            ms[j * _KK + r]
                return 0

            lax.fori_loop(0, _RPB, rbody, 0)

        for half in range(2):
            start(c_lo + half, bufs[half][0], bufs[half][1], bufs[half][2])

        def g_body(g, corr):
            for half in range(2):
                zb, sb, sem = bufs[half]
                c = c_lo + g * 2 + half
                wait(c, zb, sb, sem)
                corr = corr_update(sb, c * _VC, corr)
                compute_chunk(zb, sb)

                @pl.when(g < _CPQ // 2 - 1)
                def _(c=c, zb=zb, sb=sb, sem=sem):
                    start(c + 2, zb, sb, sem)

            return corr

        corr = lax.fori_loop(0, _CPQ // 2, g_body,
                             jnp.full((_L,), 0.0, jnp.float32))

        @pl.when(q == 0)
        def _():
            zb, sb, sem = bufs[0]
            ce = _VMAIN // _VC
            start(ce, zb, sb, sem)
            wait(ce, zb, sb, sem)
            c2 = corr_update(sb, ce * _VC, corr)
            compute_chunk(zb, sb)
            cbuf[...] = c2
            pltpu.sync_copy(cbuf, corr_hbm.at[pl.ds(wid * _L, _L)])

        @pl.when(q == 1)
        def _():
            zb, sb, sem = bufs[0]
            start(0, zb, sb, sem, zsrc=zt_hbm, ssrc=st_hbm)
            wait(0, zb, sb, sem, zsrc=zt_hbm, ssrc=st_hbm)
            c2 = corr_update(sb, _TAILV, corr)
            compute_chunk(zb, sb)
            cbuf[...] = c2
            pltpu.sync_copy(cbuf, corr_hbm.at[pl.ds(wid * _L, _L)])

        @pl.when(q >= 2)
        def _():
            cbuf[...] = corr
            pltpu.sync_copy(cbuf, corr_hbm.at[pl.ds(wid * _L, _L)])

        pltpu.sync_copy(st, cand_hbm.at[pl.ds(wid * _NST, _NST)])

    return body(s2d, y, zt3, s_tail, z_tail)


def _tc_loss(cand2d, corr2d):
    def fin(x_ref, c_ref, o_ref):
        x = x_ref[...]
        need = jnp.full((_NS * _B, 1), _KK, jnp.int32)
        ans = jnp.zeros((_NS * _B, 1), jnp.float32)
        t = jnp.full((_NS * _B, 1), jnp.inf, jnp.float32)
        for _ in range(_KK):
            masked = jnp.where(x < t, x, _NEG)
            m = jnp.max(masked, axis=1, keepdims=True)
            c = jnp.sum((x == m).astype(jnp.int32), axis=1, keepdims=True)
            hit = jnp.logical_and(need > 0, need <= c)
            ans = jnp.where(hit, m, ans)
            need = need - c
            t = m
        skp1 = jnp.zeros((_B, 1), jnp.float32)
        for j in range(_NS):
            skp1 = skp1 + ans[j * _B:(j + 1) * _B, :]
        skp1 = skp1 * jnp.float32(1.0 / _NS)
        corr = jnp.sum(c_ref[...], axis=1, keepdims=True)
        num = jnp.maximum(jnp.float32(1.0) + skp1 - corr, 0.0)
        o_ref[0, 0] = jnp.sum(num) * jnp.float32(1.0 / _B)

    return pl.pallas_call(
        fin,
        out_shape=jax.ShapeDtypeStruct((1, 1), jnp.float32),
        out_specs=pl.BlockSpec(memory_space=pltpu.SMEM),
    )(cand2d, corr2d)


def kernel(s, y, Z):
    zt3 = jnp.transpose(Z, (2, 0, 1))
    pad = _VC - _TAILN
    s_tail = jnp.pad(s[:, _TAILV:], ((0, 0), (0, pad)))
    z_tail = jnp.pad(zt3[:, :, _TAILV:], ((0, 0), (0, 0), (0, pad)),
                     constant_values=_NEG)
    cand, corr = _sc_partials(s, y.astype(jnp.int32), zt3, s_tail, z_tail)
    cand2d = cand.reshape(_NBLK, _NQ, _RPB, _NS, _KK * _L)
    cand2d = cand2d.transpose(3, 0, 2, 1, 4).reshape(_NS * _B, _NQ * _KK * _L)
    corr2d = (corr.reshape(_NBLK, _NQ, _L)[:, :, :_RPB]
              .transpose(0, 2, 1).reshape(_B, _NQ))
    return _tc_loss(cand2d, corr2d)[0, 0]
